# Initial kernel scaffold; baseline (speedup 1.0000x reference)
import jax
import jax.numpy as jnp
from jax.experimental import pallas as pl


def kernel(x, edge_index, W1, b1, Wmu, bmu, Wls, bls):
    raise NotImplementedError("write your pallas kernel here")


# speedup vs baseline: 36.7411x; 36.7411x over previous
import functools

import jax
import jax.numpy as jnp
from jax import lax
from jax.experimental import pallas as pl
from jax.experimental.pallas import tpu as pltpu
from jax.experimental.pallas import tpu_sc as plsc

NC = 2
NS = 16
NW = NC * NS
CHUNK = 128


def _ceil_to(a, m):
    return (a + m - 1) // m * m


def _mesh():
    return plsc.VectorSubcoreMesh(
        core_axis_name="c", subcore_axis_name="s",
        num_cores=NC, num_subcores=NS)


def _deg_kernel(npad, epc):
    rpt = npad // NS
    zch = rpt // CHUNK
    w = 16

    @functools.partial(
        pl.kernel,
        out_type=jax.ShapeDtypeStruct((NC, npad, w), jnp.float32),
        mesh=_mesh(),
        compiler_params=pltpu.CompilerParams(use_tc_tiling_on_sc=False),
        scratch_types=[
            pltpu.VMEM((epc, CHUNK), jnp.int32),
            pltpu.VMEM((CHUNK, w), jnp.float32),
            pltpu.VMEM((CHUNK, w), jnp.float32),
            pltpu.VMEM_SHARED((npad, w), jnp.float32),
            pltpu.SemaphoreType.DMA,
        ],
    )
    def deg(dst_hbm, out_hbm, idx_v, ones_v, zero_v, acc_sh, sem):
        cid = lax.axis_index("c")
        sid = lax.axis_index("s")
        wid = sid * NC + cid

        def fill(r, carry):
            ones_v[r, :] = jnp.full((16,), 1.0, jnp.float32)
            zero_v[r, :] = jnp.zeros((16,), jnp.float32)
            return carry
        lax.fori_loop(0, CHUNK, fill, 0)

        for z in range(zch):
            pltpu.sync_copy(zero_v,
                            acc_sh.at[pl.ds(sid * rpt + z * CHUNK, CHUNK)])
        pltpu.sync_copy(dst_hbm.at[wid], idx_v)
        plsc.subcore_barrier()

        def group(i, carry):
            for j in range(4):
                pltpu.async_copy(ones_v, acc_sh.at[idx_v.at[i * 4 + j]],
                                 sem, add=True)
            for j in range(4):
                pltpu.make_async_copy(ones_v, acc_sh.at[idx_v.at[i * 4 + j]],
                                      sem).wait()
            return carry
        lax.fori_loop(0, epc // 4, group, 0)

        plsc.subcore_barrier()
        pltpu.sync_copy(acc_sh.at[pl.ds(sid * rpt, rpt)],
                        out_hbm.at[cid, pl.ds(sid * rpt, rpt)])

    return deg


def _agg_kernel(npad, epc, width):
    rpt = npad // NS
    zch = rpt // CHUNK
    pairs = epc // 2

    @functools.partial(
        pl.kernel,
        out_type=jax.ShapeDtypeStruct((NC, npad, width), jnp.float32),
        mesh=_mesh(),
        compiler_params=pltpu.CompilerParams(use_tc_tiling_on_sc=False),
        scratch_types=[
            pltpu.VMEM((epc, CHUNK), jnp.int32),
            pltpu.VMEM((epc, CHUNK), jnp.int32),
            pltpu.VMEM((2, CHUNK, width), jnp.float32),
            pltpu.VMEM((CHUNK, width), jnp.float32),
            pltpu.VMEM_SHARED((npad, width), jnp.float32),
            pltpu.SemaphoreType.DMA,
            pltpu.SemaphoreType.DMA,
            pltpu.SemaphoreType.DMA,
            pltpu.SemaphoreType.DMA,
        ],
    )
    def agg(tab_hbm, src_hbm, dst_hbm, out_hbm,
            src_v, dst_v, rows_v, zero_v, acc_sh, gs0, gs1, ss0, ss1):
        cid = lax.axis_index("c")
        sid = lax.axis_index("s")
        wid = sid * NC + cid

        def fill(r, carry):
            for j in range(width // 16):
                zero_v[r, pl.ds(j * 16, 16)] = jnp.zeros((16,), jnp.float32)
            return carry
        lax.fori_loop(0, CHUNK, fill, 0)

        for z in range(zch):
            pltpu.sync_copy(zero_v,
                            acc_sh.at[pl.ds(sid * rpt + z * CHUNK, CHUNK)])
        pltpu.sync_copy(src_hbm.at[wid], src_v)
        pltpu.sync_copy(dst_hbm.at[wid], dst_v)
        plsc.subcore_barrier()

        gsems = (gs0, gs1)
        ssems = (ss0, ss1)

        def gstart(c, b):
            pltpu.async_copy(tab_hbm.at[src_v.at[c]], rows_v.at[b], gsems[b])

        def gwait(c, b):
            pltpu.make_async_copy(tab_hbm.at[src_v.at[c]], rows_v.at[b],
                                  gsems[b]).wait()

        def sstart(c, b):
            pltpu.async_copy(rows_v.at[b], acc_sh.at[dst_v.at[c]],
                             ssems[b], add=True)

        def swait(c, b):
            pltpu.make_async_copy(rows_v.at[b], acc_sh.at[dst_v.at[c]],
                                  ssems[b]).wait()

        gstart(0, 0)
        gstart(1, 1)

        def pair(i, carry):
            c0 = 2 * i
            c1 = c0 + 1
            gwait(c0, 0)
            sstart(c0, 0)
            gwait(c1, 1)
            sstart(c1, 1)

            @pl.when(i < pairs - 1)
            def _():
                swait(c0, 0)
                gstart(c0 + 2, 0)
                swait(c1, 1)
                gstart(c1 + 2, 1)
            return carry
        lax.fori_loop(0, pairs, pair, 0)
        swait(epc - 2, 0)
        swait(epc - 1, 1)

        plsc.subcore_barrier()
        pltpu.sync_copy(acc_sh.at[pl.ds(sid * rpt, rpt)],
                        out_hbm.at[cid, pl.ds(sid * rpt, rpt)])

    return agg


def _dinv(d0_ref, d1_ref):
    return lax.rsqrt(d0_ref[:, 0:1] + d1_ref[:, 0:1] + 1.0)


def _tca_body(x_ref, w_ref, d0_ref, d1_ref, o_ref):
    xw = jnp.dot(x_ref[...], w_ref[...], preferred_element_type=jnp.float32)
    o_ref[...] = _dinv(d0_ref, d1_ref) * xw


def _tcb_body(a0_ref, a1_ref, y1_ref, d0_ref, d1_ref, b1_ref, o_ref):
    dinv = _dinv(d0_ref, d1_ref)
    u = a0_ref[...] + a1_ref[...] + y1_ref[...]
    h = jnp.maximum(dinv * u + b1_ref[...], 0.0)
    o_ref[...] = dinv * h


def _tcc_body(a0_ref, a1_ref, y2_ref, d0_ref, d1_ref, wc_ref, bc_ref, o_ref):
    dinv = _dinv(d0_ref, d1_ref)
    z = dinv * (a0_ref[...] + a1_ref[...] + y2_ref[...])
    o_ref[...] = (jnp.dot(z, wc_ref[...], preferred_element_type=jnp.float32)
                  + bc_ref[...])


def kernel(x, edge_index, W1, b1, Wmu, bmu, Wls, bls):
    n, d_in = x.shape
    h_dim = W1.shape[1]
    out_dim = Wmu.shape[1]
    e = edge_index.shape[1]

    npad = _ceil_to(n + CHUNK, NS * CHUNK)
    epw = _ceil_to(-(-e // NW), 4 * CHUNK)
    epc = epw // CHUNK
    epad = epw * NW

    pad_idx = (n + (jnp.arange(epad - e, dtype=jnp.int32) % CHUNK))
    srcp = jnp.concatenate([edge_index[0], pad_idx]).reshape(NW, epc, CHUNK)
    dstp = jnp.concatenate([edge_index[1], pad_idx]).reshape(NW, epc, CHUNK)
    x_pad = jnp.zeros((npad, d_in), x.dtype).at[:n].set(x)

    degp = _deg_kernel(npad, epc)(dstp)
    d0 = degp[0]
    d1 = degp[1]

    br = 2048
    grid = (npad // br,)
    row_spec = lambda width: pl.BlockSpec((br, width), lambda i: (i, 0))
    full_spec = lambda shape: pl.BlockSpec(shape, lambda i: (0, 0))

    y1 = pl.pallas_call(
        _tca_body,
        grid=grid,
        in_specs=[row_spec(d_in), full_spec((d_in, h_dim)),
                  row_spec(16), row_spec(16)],
        out_specs=row_spec(h_dim),
        out_shape=jax.ShapeDtypeStruct((npad, h_dim), jnp.float32),
    )(x_pad, W1, d0, d1)

    agg = _agg_kernel(npad, epc, h_dim)
    a1p = agg(y1, srcp, dstp)

    y2 = pl.pallas_call(
        _tcb_body,
        grid=grid,
        in_specs=[row_spec(h_dim), row_spec(h_dim), row_spec(h_dim),
                  row_spec(16), row_spec(16), full_spec((1, h_dim))],
        out_specs=row_spec(h_dim),
        out_shape=jax.ShapeDtypeStruct((npad, h_dim), jnp.float32),
    )(a1p[0], a1p[1], y1, d0, d1, b1.reshape(1, h_dim))

    a2p = agg(y2, srcp, dstp)

    wc = jnp.concatenate([Wmu, Wls], axis=1)
    bc = jnp.concatenate([bmu, bls]).reshape(1, 2 * out_dim)
    out = pl.pallas_call(
        _tcc_body,
        grid=grid,
        in_specs=[row_spec(h_dim), row_spec(h_dim), row_spec(h_dim),
                  row_spec(16), row_spec(16),
                  full_spec((h_dim, 2 * out_dim)), full_spec((1, 2 * out_dim))],
        out_specs=row_spec(2 * out_dim),
        out_shape=jax.ShapeDtypeStruct((npad, 2 * out_dim), jnp.float32),
    )(a2p[0], a2p[1], y2, d0, d1, wc, bc)

    return out[:n, :out_dim], out[:n, out_dim:]


# speedup vs baseline: 44.0457x; 1.1988x over previous
import functools

import jax
import jax.numpy as jnp
from jax import lax
from jax.experimental import pallas as pl
from jax.experimental.pallas import tpu as pltpu
from jax.experimental.pallas import tpu_sc as plsc

NC = 2
NS = 16
NW = NC * NS
CHUNK = 128


def _ceil_to(a, m):
    return (a + m - 1) // m * m


def _mesh():
    return plsc.VectorSubcoreMesh(
        core_axis_name="c", subcore_axis_name="s",
        num_cores=NC, num_subcores=NS)


def _deg_kernel(npad, epc):
    rpt = npad // NS
    zch = rpt // CHUNK
    w = 16

    @functools.partial(
        pl.kernel,
        out_type=jax.ShapeDtypeStruct((NC, npad, w), jnp.float32),
        mesh=_mesh(),
        compiler_params=pltpu.CompilerParams(use_tc_tiling_on_sc=False),
        scratch_types=[
            pltpu.VMEM((epc, CHUNK), jnp.int32),
            pltpu.VMEM((CHUNK, w), jnp.float32),
            pltpu.VMEM((CHUNK, w), jnp.float32),
            pltpu.VMEM_SHARED((npad, w), jnp.float32),
            pltpu.SemaphoreType.DMA,
        ],
    )
    def deg(dst_hbm, out_hbm, idx_v, ones_v, zero_v, acc_sh, sem):
        cid = lax.axis_index("c")
        sid = lax.axis_index("s")
        wid = sid * NC + cid

        def fill(r, carry):
            ones_v[r, :] = jnp.full((16,), 1.0, jnp.float32)
            zero_v[r, :] = jnp.zeros((16,), jnp.float32)
            return carry
        lax.fori_loop(0, CHUNK, fill, 0)

        for z in range(zch):
            pltpu.sync_copy(zero_v,
                            acc_sh.at[pl.ds(sid * rpt + z * CHUNK, CHUNK)])
        pltpu.sync_copy(dst_hbm.at[wid], idx_v)
        plsc.subcore_barrier()

        def group(i, carry):
            for j in range(4):
                pltpu.async_copy(ones_v, acc_sh.at[idx_v.at[i * 4 + j]],
                                 sem, add=True)
            for j in range(4):
                pltpu.make_async_copy(ones_v, acc_sh.at[idx_v.at[i * 4 + j]],
                                      sem).wait()
            return carry
        lax.fori_loop(0, epc // 4, group, 0)

        plsc.subcore_barrier()
        pltpu.sync_copy(acc_sh.at[pl.ds(sid * rpt, rpt)],
                        out_hbm.at[cid, pl.ds(sid * rpt, rpt)])

    return deg


def _agg_kernel(npad, epc, width):
    rpt = npad // NS
    zch = rpt // CHUNK
    nbuf = 4
    rounds = epc // nbuf

    @functools.partial(
        pl.kernel,
        out_type=jax.ShapeDtypeStruct((NC, npad, width), jnp.float32),
        mesh=_mesh(),
        compiler_params=pltpu.CompilerParams(use_tc_tiling_on_sc=False),
        scratch_types=[
            pltpu.VMEM((epc, CHUNK), jnp.int32),
            pltpu.VMEM((epc, CHUNK), jnp.int32),
            pltpu.VMEM((nbuf, CHUNK, width), jnp.float32),
            pltpu.VMEM((CHUNK, width), jnp.float32),
            pltpu.VMEM_SHARED((npad, width), jnp.float32),
            pltpu.SemaphoreType.DMA,
        ] + [pltpu.SemaphoreType.DMA] * (2 * nbuf),
    )
    def agg(tab_hbm, src_hbm, dst_hbm, out_hbm,
            src_v, dst_v, rows_v, zero_v, acc_sh, lsem, *sems):
        cid = lax.axis_index("c")
        sid = lax.axis_index("s")
        wid = sid * NC + cid
        gsems = sems[:nbuf]
        ssems = sems[nbuf:]

        pltpu.async_copy(src_hbm.at[wid], src_v, lsem)
        pltpu.async_copy(dst_hbm.at[wid], dst_v, lsem)

        def fill(r, carry):
            for j in range(width // 16):
                zero_v[r, pl.ds(j * 16, 16)] = jnp.zeros((16,), jnp.float32)
            return carry
        lax.fori_loop(0, CHUNK, fill, 0)

        for z in range(zch):
            pltpu.sync_copy(zero_v,
                            acc_sh.at[pl.ds(sid * rpt + z * CHUNK, CHUNK)])
        pltpu.make_async_copy(src_hbm.at[wid], src_v, lsem).wait()
        pltpu.make_async_copy(dst_hbm.at[wid], dst_v, lsem).wait()
        plsc.subcore_barrier()

        def gstart(c, b):
            pltpu.async_copy(tab_hbm.at[src_v.at[c]], rows_v.at[b], gsems[b])

        def gwait(c, b):
            pltpu.make_async_copy(tab_hbm.at[src_v.at[c]], rows_v.at[b],
                                  gsems[b]).wait()

        def sstart(c, b):
            pltpu.async_copy(rows_v.at[b], acc_sh.at[dst_v.at[c]],
                             ssems[b], add=True)

        def swait(c, b):
            pltpu.make_async_copy(rows_v.at[b], acc_sh.at[dst_v.at[c]],
                                  ssems[b]).wait()

        for b in range(nbuf):
            gstart(b, b)

        def round_body(i, carry):
            for b in range(nbuf):
                c = nbuf * i + b
                gwait(c, b)
                sstart(c, b)
            for b in range(nbuf):
                c = nbuf * i + b

                @pl.when(c + nbuf < epc)
                def _():
                    swait(c, b)
                    gstart(c + nbuf, b)
            return carry
        lax.fori_loop(0, rounds, round_body, 0)
        for b in range(nbuf):
            swait(epc - nbuf + b, b)

        plsc.subcore_barrier()
        pltpu.sync_copy(acc_sh.at[pl.ds(sid * rpt, rpt)],
                        out_hbm.at[cid, pl.ds(sid * rpt, rpt)])

    return agg


def _dinv(d0_ref, d1_ref):
    return lax.rsqrt(d0_ref[:, 0:1] + d1_ref[:, 0:1] + 1.0)


def _tca_body(x_ref, w_ref, d0_ref, d1_ref, o_ref):
    xw = jnp.dot(x_ref[...], w_ref[...], preferred_element_type=jnp.float32)
    o_ref[...] = _dinv(d0_ref, d1_ref) * xw


def _tcb_body(a0_ref, a1_ref, y1_ref, d0_ref, d1_ref, b1_ref, o_ref):
    dinv = _dinv(d0_ref, d1_ref)
    u = a0_ref[...] + a1_ref[...] + y1_ref[...]
    h = jnp.maximum(dinv * u + b1_ref[...], 0.0)
    o_ref[...] = dinv * h


def _tcc_body(a0_ref, a1_ref, y2_ref, d0_ref, d1_ref, wc_ref, bc_ref, o_ref):
    dinv = _dinv(d0_ref, d1_ref)
    z = dinv * (a0_ref[...] + a1_ref[...] + y2_ref[...])
    o_ref[...] = (jnp.dot(z, wc_ref[...], preferred_element_type=jnp.float32)
                  + bc_ref[...])


def kernel(x, edge_index, W1, b1, Wmu, bmu, Wls, bls):
    n, d_in = x.shape
    h_dim = W1.shape[1]
    out_dim = Wmu.shape[1]
    e = edge_index.shape[1]

    npad = _ceil_to(n + CHUNK, NS * CHUNK)
    epw = _ceil_to(-(-e // NW), 4 * CHUNK)
    epc = epw // CHUNK
    epad = epw * NW

    pad_idx = (n + (jnp.arange(epad - e, dtype=jnp.int32) % CHUNK))
    srcp = jnp.concatenate([edge_index[0], pad_idx]).reshape(NW, epc, CHUNK)
    dstp = jnp.concatenate([edge_index[1], pad_idx]).reshape(NW, epc, CHUNK)
    x_pad = jnp.zeros((npad, d_in), x.dtype).at[:n].set(x)

    degp = _deg_kernel(npad, epc)(dstp)
    d0 = degp[0]
    d1 = degp[1]

    br = 2048
    grid = (npad // br,)
    row_spec = lambda width: pl.BlockSpec((br, width), lambda i: (i, 0))
    full_spec = lambda shape: pl.BlockSpec(shape, lambda i: (0, 0))

    y1 = pl.pallas_call(
        _tca_body,
        grid=grid,
        in_specs=[row_spec(d_in), full_spec((d_in, h_dim)),
                  row_spec(16), row_spec(16)],
        out_specs=row_spec(h_dim),
        out_shape=jax.ShapeDtypeStruct((npad, h_dim), jnp.float32),
    )(x_pad, W1, d0, d1)

    agg = _agg_kernel(npad, epc, h_dim)
    a1p = agg(y1, srcp, dstp)

    y2 = pl.pallas_call(
        _tcb_body,
        grid=grid,
        in_specs=[row_spec(h_dim), row_spec(h_dim), row_spec(h_dim),
                  row_spec(16), row_spec(16), full_spec((1, h_dim))],
        out_specs=row_spec(h_dim),
        out_shape=jax.ShapeDtypeStruct((npad, h_dim), jnp.float32),
    )(a1p[0], a1p[1], y1, d0, d1, b1.reshape(1, h_dim))

    a2p = agg(y2, srcp, dstp)

    wc = jnp.concatenate([Wmu, Wls], axis=1)
    bc = jnp.concatenate([bmu, bls]).reshape(1, 2 * out_dim)
    out = pl.pallas_call(
        _tcc_body,
        grid=grid,
        in_specs=[row_spec(h_dim), row_spec(h_dim), row_spec(h_dim),
                  row_spec(16), row_spec(16),
                  full_spec((h_dim, 2 * out_dim)), full_spec((1, 2 * out_dim))],
        out_specs=row_spec(2 * out_dim),
        out_shape=jax.ShapeDtypeStruct((npad, 2 * out_dim), jnp.float32),
    )(a2p[0], a2p[1], y2, d0, d1, wc, bc)

    return out[:n, :out_dim], out[:n, out_dim:]
